# Initial kernel scaffold; baseline (speedup 1.0000x reference)
import jax
import jax.numpy as jnp
from jax.experimental import pallas as pl


def kernel(x, edge_index, edge_embedding, Wq, bq, Wk, bk, Wv, bv, We, Wskip, bskip, ln1_g, ln1_b, W1, b1, W2, b2, ln2_g, ln2_b):
    raise NotImplementedError("write your pallas kernel here")


# speedup vs baseline: 15.2940x; 15.2940x over previous
import functools
import math

import jax
import jax.numpy as jnp
from jax import lax
from jax.experimental import pallas as pl
from jax.experimental.pallas import tpu as pltpu
from jax.experimental.pallas import tpu_sc as plsc

N = 10000
E = 160000
D = 256
H = 4
C = D // H

NSUB = 16
EB = 40
E_PER_SUB = E // NSUB
NBLK = E_PER_SUB // EB
N_PAD = 10240
N_PER_SUB = N_PAD // NSUB
ZROWS = 128


def _proj_body(x_ref, w_ref, b_ref, q0, q1, k0, k1, v0, v1, skip):
    acc = jnp.dot(x_ref[...], w_ref[...], preferred_element_type=jnp.float32)
    acc = acc + b_ref[...]
    q0[...] = acc[:, 0:128]
    q1[...] = acc[:, 128:256]
    k0[...] = acc[:, 256:384]
    k1[...] = acc[:, 384:512]
    v0[...] = acc[:, 512:640]
    v1[...] = acc[:, 640:768]
    skip[...] = acc[:, 768:1024]


def _proj(x, w, b):
    blk = 2000
    grid = N // blk
    half = jax.ShapeDtypeStruct((N, 128), jnp.float32)
    return pl.pallas_call(
        _proj_body,
        grid=(grid,),
        in_specs=[
            pl.BlockSpec((blk, D), lambda i: (i, 0)),
            pl.BlockSpec((D, 4 * D), lambda i: (0, 0)),
            pl.BlockSpec((1, 4 * D), lambda i: (0, 0)),
        ],
        out_specs=[pl.BlockSpec((blk, 128), lambda i: (i, 0))] * 6
        + [pl.BlockSpec((blk, D), lambda i: (i, 0))],
        out_shape=[half] * 6 + [jax.ShapeDtypeStruct((N, D), jnp.float32)],
    )(x, w, b)


def _eproj_body(emb_ref, w_ref, e0, e1):
    acc = jnp.dot(emb_ref[...], w_ref[...], preferred_element_type=jnp.float32)
    e0[...] = acc[:, 0:128]
    e1[...] = acc[:, 128:256]


def _eproj(emb, we):
    blk = 2000
    grid = E // blk
    half = jax.ShapeDtypeStruct((E, 128), jnp.float32)
    return pl.pallas_call(
        _eproj_body,
        grid=(grid,),
        in_specs=[
            pl.BlockSpec((blk, D), lambda i: (i, 0)),
            pl.BlockSpec((D, D), lambda i: (0, 0)),
        ],
        out_specs=[pl.BlockSpec((blk, 128), lambda i: (i, 0))] * 2,
        out_shape=[half, half],
    )(emb, we)


def _sc_core_work(s, src_h, dst_h, qt, kt, vt, et, num_out, den_out,
                  sidx, didx, qb, kb, vb, eb, exb,
                  acc, dacc, sem0, sem1, sem2, sem3):
    zeros16 = jnp.zeros((16,), jnp.float32)

    def _zrow(i, _):
        for cc in range(8):
            qb[i, pl.ds(cc * 16, 16)] = zeros16
        exb[i, :] = zeros16
        return 0

    lax.fori_loop(0, EB, _zrow, 0)

    for t in range(N_PER_SUB // EB):
        pltpu.sync_copy(qb, acc.at[pl.ds(s * N_PER_SUB + t * EB, EB)])
        pltpu.sync_copy(exb, dacc.at[pl.ds(s * N_PER_SUB + t * EB, EB)])

    plsc.subcore_barrier()

    lane = lax.iota(jnp.int32, 16)
    m0 = lane == 0
    m1 = lane == 1

    def _block(j, _):
        base = s * E_PER_SUB + j * EB
        pltpu.sync_copy(src_h.at[pl.ds(base, EB)], sidx)
        pltpu.sync_copy(dst_h.at[pl.ds(base, EB)], didx)
        cq = pltpu.async_copy(qt.at[didx], qb, sem0)
        ck = pltpu.async_copy(kt.at[sidx], kb, sem1)
        cv = pltpu.async_copy(vt.at[sidx], vb, sem2)
        ce = pltpu.async_copy(et.at[pl.ds(base, EB)], eb, sem3)
        cq.wait()
        ck.wait()
        cv.wait()
        ce.wait()

        def _edge(i, _):
            ev = [eb[i, pl.ds(cc * 16, 16)] for cc in range(8)]
            exs = []
            for h in range(2):
                off = h * 64
                dot = jnp.zeros((16,), jnp.float32)
                for cc in range(4):
                    qv = qb[i, pl.ds(off + cc * 16, 16)]
                    kv = kb[i, pl.ds(off + cc * 16, 16)]
                    dot = dot + qv * (kv + ev[4 * h + cc])
                alpha = jnp.sum(dot) * 0.125
                exv = jnp.exp(jnp.broadcast_to(alpha, (16,)))
                for cc in range(4):
                    sl = pl.ds(off + cc * 16, 16)
                    vb[i, sl] = (vb[i, sl] + ev[4 * h + cc]) * exv
                exs.append(exv)
            exrow = jnp.where(m0, exs[0], jnp.where(m1, exs[1], zeros16))
            exb[i, :] = exrow
            return 0

        lax.fori_loop(0, EB, _edge, 0)

        pltpu.sync_copy(vb, acc.at[didx], add=True)
        pltpu.sync_copy(exb, dacc.at[didx], add=True)
        return 0

    lax.fori_loop(0, NBLK, _block, 0)

    plsc.subcore_barrier()

    lo = s * N_PER_SUB
    pltpu.sync_copy(acc.at[pl.ds(lo, N_PER_SUB)], num_out.at[pl.ds(lo, N_PER_SUB)])
    pltpu.sync_copy(dacc.at[pl.ds(lo, N_PER_SUB)], den_out.at[pl.ds(lo, N_PER_SUB)])


def _sc_edge(src, dst, q0, k0, v0, e0, q1, k1, v1, e1):
    mesh = plsc.VectorSubcoreMesh(core_axis_name="c", subcore_axis_name="s")

    @functools.partial(
        pl.kernel,
        mesh=mesh,
        compiler_params=pltpu.CompilerParams(
            needs_layout_passes=False, use_tc_tiling_on_sc=False),
        out_type=[
            jax.ShapeDtypeStruct((N_PAD, 128), jnp.float32),
            jax.ShapeDtypeStruct((N_PAD, 128), jnp.float32),
            jax.ShapeDtypeStruct((N_PAD, 16), jnp.float32),
            jax.ShapeDtypeStruct((N_PAD, 16), jnp.float32),
        ],
        scratch_types=[
            pltpu.VMEM((EB,), jnp.int32),
            pltpu.VMEM((EB,), jnp.int32),
            pltpu.VMEM((EB, 128), jnp.float32),
            pltpu.VMEM((EB, 128), jnp.float32),
            pltpu.VMEM((EB, 128), jnp.float32),
            pltpu.VMEM((EB, 128), jnp.float32),
            pltpu.VMEM((EB, 16), jnp.float32),
            pltpu.VMEM_SHARED((N_PAD, 128), jnp.float32),
            pltpu.VMEM_SHARED((N_PAD, 16), jnp.float32),
            pltpu.SemaphoreType.DMA,
            pltpu.SemaphoreType.DMA,
            pltpu.SemaphoreType.DMA,
            pltpu.SemaphoreType.DMA,
        ],
    )
    def k(src_h, dst_h, q0_h, k0_h, v0_h, e0_h, q1_h, k1_h, v1_h, e1_h,
          num0, num1, den0, den1, *scr):
        c = lax.axis_index("c")
        s = lax.axis_index("s")

        @pl.when(c == 0)
        def _():
            _sc_core_work(s, src_h, dst_h, q0_h, k0_h, v0_h, e0_h, num0, den0, *scr)

        @pl.when(c == 1)
        def _():
            _sc_core_work(s, src_h, dst_h, q1_h, k1_h, v1_h, e1_h, num1, den1, *scr)

    return k(src, dst, q0, k0, v0, e0, q1, k1, v1, e1)


def _final_body(x_ref, n0_ref, n1_ref, d0_ref, d1_ref, skip_ref,
                g1_ref, b1n_ref, w1_ref, bf1_ref, w2_ref, bf2_ref,
                g2_ref, b2n_ref, out_ref):
    rows = x_ref.shape[0]
    eps = 1e-16

    d0 = d0_ref[...]
    d1 = d1_ref[...]
    div0 = jnp.concatenate(
        [jnp.broadcast_to(d0[:, 0:1], (rows, 64)),
         jnp.broadcast_to(d0[:, 1:2], (rows, 64))], axis=1)
    div1 = jnp.concatenate(
        [jnp.broadcast_to(d1[:, 0:1], (rows, 64)),
         jnp.broadcast_to(d1[:, 1:2], (rows, 64))], axis=1)
    agg = jnp.concatenate(
        [n0_ref[...] / (div0 + eps), n1_ref[...] / (div1 + eps)], axis=1)

    x = x_ref[...]
    attended = agg + skip_ref[...]
    pre = x + attended
    mu = jnp.mean(pre, axis=1, keepdims=True)
    var = jnp.mean((pre - mu) ** 2, axis=1, keepdims=True)
    h = (pre - mu) * lax.rsqrt(var + 1e-5) * g1_ref[...] + b1n_ref[...]

    t = jnp.dot(h, w1_ref[...], preferred_element_type=jnp.float32) + bf1_ref[...]
    t = jnp.where(t >= 0, t, 0.01 * t)
    ffn = jnp.dot(t, w2_ref[...], preferred_element_type=jnp.float32) + bf2_ref[...]

    pre2 = h + ffn
    mu2 = jnp.mean(pre2, axis=1, keepdims=True)
    var2 = jnp.mean((pre2 - mu2) ** 2, axis=1, keepdims=True)
    out_ref[...] = (pre2 - mu2) * lax.rsqrt(var2 + 1e-5) * g2_ref[...] + b2n_ref[...]


def _final(x, n0, n1, d0, d1, skip, g1, b1n, w1, bf1, w2, bf2, g2, b2n):
    blk = 2000
    grid = N // blk
    return pl.pallas_call(
        _final_body,
        grid=(grid,),
        in_specs=[
            pl.BlockSpec((blk, D), lambda i: (i, 0)),
            pl.BlockSpec((blk, 128), lambda i: (i, 0)),
            pl.BlockSpec((blk, 128), lambda i: (i, 0)),
            pl.BlockSpec((blk, 16), lambda i: (i, 0)),
            pl.BlockSpec((blk, 16), lambda i: (i, 0)),
            pl.BlockSpec((blk, D), lambda i: (i, 0)),
            pl.BlockSpec((1, D), lambda i: (0, 0)),
            pl.BlockSpec((1, D), lambda i: (0, 0)),
            pl.BlockSpec((D, 2 * D), lambda i: (0, 0)),
            pl.BlockSpec((1, 2 * D), lambda i: (0, 0)),
            pl.BlockSpec((2 * D, D), lambda i: (0, 0)),
            pl.BlockSpec((1, D), lambda i: (0, 0)),
            pl.BlockSpec((1, D), lambda i: (0, 0)),
            pl.BlockSpec((1, D), lambda i: (0, 0)),
        ],
        out_specs=pl.BlockSpec((blk, D), lambda i: (i, 0)),
        out_shape=jax.ShapeDtypeStruct((N, D), jnp.float32),
    )(x, n0, n1, d0, d1, skip, g1, b1n, w1, bf1, w2, bf2, g2, b2n)


def kernel(x, edge_index, edge_embedding, Wq, bq, Wk, bk, Wv, bv, We,
           Wskip, bskip, ln1_g, ln1_b, W1, b1, W2, b2, ln2_g, ln2_b):
    ei = edge_index.astype(jnp.int32)
    src = ei[0]
    dst = ei[1]
    w = jnp.concatenate([Wq, Wk, Wv, Wskip], axis=1)
    b = jnp.concatenate([bq, bk, bv, bskip])[None, :]

    q0, q1, k0, k1, v0, v1, skip = _proj(x, w, b)
    e0, e1 = _eproj(edge_embedding, We)
    num0, num1, den0, den1 = _sc_edge(src, dst, q0, k0, v0, e0, q1, k1, v1, e1)
    out = _final(x, num0, num1, den0, den1, skip,
                 ln1_g[None, :], ln1_b[None, :], W1, b1[None, :],
                 W2, b2[None, :], ln2_g[None, :], ln2_b[None, :])
    return out
